# baseline (speedup 1.0000x reference)
import functools

import jax
import jax.numpy as jnp
from jax import lax
from jax.experimental import pallas as pl
from jax.experimental.pallas import tpu as pltpu
from jax.experimental.pallas import tpu_sc as plsc

VOCAB = 100000
DIM = 64
BATCH = 4096
LENGTH = 200
N = BATCH * LENGTH

NC = 2
NS = 16
NW = NC * NS
WR = N // NW
CW = 128
STEPS = WR // CW
NBUF = 8
GROUPS = STEPS // NBUF

_mesh = plsc.VectorSubcoreMesh(core_axis_name="c", subcore_axis_name="s")


@functools.partial(
    pl.kernel,
    mesh=_mesh,
    out_type=jax.ShapeDtypeStruct((N, DIM), jnp.float32),
    scratch_types=(
        [pltpu.VMEM((WR,), jnp.int32)]
        + [pltpu.VMEM((CW, DIM), jnp.float32) for _ in range(NBUF)]
        + [pltpu.SemaphoreType.DMA for _ in range(2 * NBUF)]
    ),
    compiler_params=pltpu.CompilerParams(use_tc_tiling_on_sc=False),
)
def _gather_kernel(idx_hbm, table_hbm, out_hbm, idx_all, *bufs_and_sems):
    rows = bufs_and_sems[0:NBUF]
    g_sems = bufs_and_sems[NBUF:2 * NBUF]
    o_sems = bufs_and_sems[2 * NBUF:]

    wid = lax.axis_index("s") * NC + lax.axis_index("c")
    base = wid * WR

    pltpu.sync_copy(idx_hbm.at[pl.ds(base, WR)], idx_all)

    def gather(b, chunk_i):
        return pltpu.make_async_copy(
            table_hbm.at[idx_all.at[pl.ds(chunk_i * CW, CW)]], rows[b],
            g_sems[b])

    def out_copy(b, chunk_i):
        return pltpu.make_async_copy(
            rows[b], out_hbm.at[pl.ds(base + chunk_i * CW, CW)], o_sems[b])

    for b in range(NBUF):
        gather(b, b).start()

    def group(g, carry):
        for b in range(NBUF):
            i = g * NBUF + b
            gather(b, i).wait()
            out_copy(b, i).start()
        for b in range(NBUF):
            i_next = (g + 1) * NBUF + b

            @pl.when(i_next < STEPS)
            def _():
                out_copy(b, i_next - NBUF).wait()
                gather(b, i_next).start()

        return carry

    lax.fori_loop(0, GROUPS, group, 0)

    for b in range(NBUF):
        out_copy(b, STEPS - NBUF + b).wait()


TB = 1024
KB = 5
KH = LENGTH // 2


def _transpose_body(x_ref, o_ref):
    for t in range(TB // 128):
        x3 = x_ref[pl.ds(t * 128, 128), :].reshape(128, KB, 128)
        o_ref[:, :, pl.ds(t * 128, 128)] = (
            x3.transpose(1, 2, 0).reshape(2 * KB, DIM, 128))


_transpose_kernel = pl.pallas_call(
    _transpose_body,
    grid=(BATCH // TB, KH // KB),
    in_specs=[pl.BlockSpec((TB, KB * 128), lambda i, j: (i, j))],
    out_specs=pl.BlockSpec((2 * KB, DIM, TB), lambda i, j: (j, 0, i)),
    out_shape=jax.ShapeDtypeStruct((LENGTH, DIM, BATCH), jnp.float32),
)


def kernel(wids, wordemb):
    flat = wids.reshape(-1).astype(jnp.int32)
    rows = _gather_kernel(flat, wordemb)
    wide = rows.reshape(BATCH, LENGTH * DIM)
    ldb = _transpose_kernel(wide)
    return ldb.transpose(2, 0, 1)


# speedup vs baseline: 8.7218x; 8.7218x over previous
import functools

import jax
import jax.numpy as jnp
from jax import lax
from jax.experimental import pallas as pl
from jax.experimental.pallas import tpu as pltpu
from jax.experimental.pallas import tpu_sc as plsc

VOCAB = 100000
DIM = 64
BATCH = 4096
LENGTH = 200
N = BATCH * LENGTH

NSLAB = 4
BSLAB = BATCH // NSLAB
NSL = N // NSLAB

NC = 2
NS = 16
NW = NC * NS
WR = NSL // NW
CW = 128
STEPS = WR // CW
NBUF = 5
GROUPS = STEPS // NBUF

_mesh = plsc.VectorSubcoreMesh(core_axis_name="c", subcore_axis_name="s")


@functools.partial(
    pl.kernel,
    mesh=_mesh,
    out_type=jax.ShapeDtypeStruct((NSL, DIM), jnp.float32),
    scratch_types=(
        [pltpu.VMEM((WR,), jnp.int32)]
        + [pltpu.VMEM((CW, DIM), jnp.float32) for _ in range(NBUF)]
        + [pltpu.SemaphoreType.DMA for _ in range(2 * NBUF)]
    ),
    compiler_params=pltpu.CompilerParams(use_tc_tiling_on_sc=False),
)
def _gather_kernel(idx_hbm, table_hbm, out_hbm, idx_all, *bufs_and_sems):
    rows = bufs_and_sems[0:NBUF]
    g_sems = bufs_and_sems[NBUF:2 * NBUF]
    o_sems = bufs_and_sems[2 * NBUF:]

    wid = lax.axis_index("s") * NC + lax.axis_index("c")
    base = wid * WR

    pltpu.sync_copy(idx_hbm.at[pl.ds(base, WR)], idx_all)

    def gather(b, chunk_i):
        return pltpu.make_async_copy(
            table_hbm.at[idx_all.at[pl.ds(chunk_i * CW, CW)]], rows[b],
            g_sems[b])

    def out_copy(b, chunk_i):
        return pltpu.make_async_copy(
            rows[b], out_hbm.at[pl.ds(base + chunk_i * CW, CW)], o_sems[b])

    for b in range(NBUF):
        gather(b, b).start()

    def group(g, carry):
        for b in range(NBUF):
            i = g * NBUF + b
            gather(b, i).wait()
            out_copy(b, i).start()
        for b in range(NBUF):
            i_next = (g + 1) * NBUF + b

            @pl.when(i_next < STEPS)
            def _():
                out_copy(b, i_next - NBUF).wait()
                gather(b, i_next).start()

        return carry

    lax.fori_loop(0, GROUPS, group, 0)

    for b in range(NBUF):
        out_copy(b, STEPS - NBUF + b).wait()


TB = 128
TROWS = TB * LENGTH // 2
KH = LENGTH // 2
TGRID = BSLAB // TB


def _transpose_body(x_ref, o_ref):
    x3 = x_ref[...].reshape(TB, KH, 128)
    o_ref[...] = x3.transpose(1, 2, 0).reshape(LENGTH, DIM, TB)


def _transpose_first(x_ref, o_ref):
    _transpose_body(x_ref, o_ref)


def _slab_transpose(s):
    body = _transpose_first if s == 0 else _transpose_body
    in_specs = [pl.BlockSpec((TROWS, 128), lambda i: (i, 0))]
    extra = ()
    if s > 0:
        in_specs.append(pl.BlockSpec(memory_space=pl.ANY))

        def body(x_ref, z_ref, o_ref):
            del z_ref
            _transpose_body(x_ref, o_ref)

        extra = (dict(input_output_aliases={1: 0}),)
    return pl.pallas_call(
        body,
        grid=(TGRID,),
        in_specs=in_specs,
        out_specs=pl.BlockSpec(
            (LENGTH, DIM, TB), lambda i, s=s: (0, 0, s * TGRID + i)),
        out_shape=jax.ShapeDtypeStruct((LENGTH, DIM, BATCH), jnp.float32),
        **(extra[0] if extra else {}),
    )


_slab_transposes = [_slab_transpose(s) for s in range(NSLAB)]


def kernel(wids, wordemb):
    flat = wids.reshape(-1).astype(jnp.int32)
    z = None
    for s in range(NSLAB):
        rows = _gather_kernel(flat[s * NSL:(s + 1) * NSL], wordemb)
        wide = rows.reshape(NSL // 2, 2 * DIM)
        if s == 0:
            z = _slab_transposes[0](wide)
        else:
            z = _slab_transposes[s](wide, z)
    return z.transpose(2, 0, 1)
